# baseline (speedup 1.0000x reference)
import functools

import jax
import jax.numpy as jnp
from jax import lax
from jax.experimental import pallas as pl
from jax.experimental.pallas import tpu as pltpu
from jax.experimental.pallas import tpu_sc as plsc

N = 10000
E = 320000
D = 128
NC = 2
NS = 16
NW = NC * NS
RANGE = 320
NPAD = NW * RANGE
CAP = 11264
SCH = 2000
NSCAN = E // SCH
CH = 128


def _sc_spmm(features, src, dst, vals):
    mesh = plsc.VectorSubcoreMesh(core_axis_name="c", subcore_axis_name="s")

    @functools.partial(
        pl.kernel,
        out_type=jax.ShapeDtypeStruct((NPAD, D), jnp.float32),
        mesh=mesh,
        scratch_types=[
            pltpu.VMEM((SCH,), jnp.int32),
            pltpu.VMEM((SCH,), jnp.int32),
            pltpu.VMEM((SCH,), jnp.int32),
            pltpu.VMEM((SCH,), jnp.int32),
            pltpu.VMEM((SCH,), jnp.float32),
            pltpu.VMEM((SCH,), jnp.float32),
            pltpu.VMEM((CAP,), jnp.int32),
            pltpu.VMEM((CAP,), jnp.int32),
            pltpu.VMEM((CAP,), jnp.float32),
            pltpu.VMEM((CH, D), jnp.float32),
            pltpu.VMEM((CH, D), jnp.float32),
            pltpu.VMEM((RANGE, D), jnp.float32),
            pltpu.SemaphoreType.DMA,
            pltpu.SemaphoreType.DMA,
        ],
        compiler_params=pltpu.CompilerParams(needs_layout_passes=False),
    )
    def k(feat_hbm, src_hbm, dst_hbm, vals_hbm, out_hbm,
          dscan0, dscan1, sscan0, sscan1, vscan0, vscan1,
          dl_list, s_list, v_list, rows0, rows1, acc, sem0, sem1):
        dscans = (dscan0, dscan1)
        sscans = (sscan0, sscan1)
        vscans = (vscan0, vscan1)
        rowss = (rows0, rows1)
        c = lax.axis_index("c")
        s = lax.axis_index("s")
        wid = s * NC + c
        lo = wid * RANGE
        lov = jnp.full((16,), lo, jnp.int32)
        rngv = jnp.full((16,), RANGE, jnp.uint32)
        iota = lax.iota(jnp.int32, 16)
        zeros16 = jnp.zeros((16,), jnp.float32)
        sems = (sem0, sem1)

        @plsc.parallel_loop(0, RANGE, unroll=4)
        def _(r):
            for d in range(D // 16):
                acc[r, pl.ds(d * 16, 16)] = zeros16

        def scan_issue(ci, b):
            off = ci * SCH
            pltpu.async_copy(dst_hbm.at[pl.ds(off, SCH)], dscans[b],
                             sems[b])
            pltpu.async_copy(src_hbm.at[pl.ds(off, SCH)], sscans[b],
                             sems[b])
            pltpu.async_copy(vals_hbm.at[pl.ds(off, SCH)], vscans[b],
                             sems[b])

        def scan_drain(ci, b):
            off = ci * SCH
            pltpu.make_async_copy(dst_hbm.at[pl.ds(off, SCH)], dscans[b],
                                  sems[b]).wait()
            pltpu.make_async_copy(src_hbm.at[pl.ds(off, SCH)], sscans[b],
                                  sems[b]).wait()
            pltpu.make_async_copy(vals_hbm.at[pl.ds(off, SCH)], vscans[b],
                                  sems[b]).wait()

        def scan_compute(b, cntv0):
            def scan_vec(vi, cntv):
                sl = pl.ds(vi * 16, 16)
                d16 = dscans[b][sl]
                dl16 = d16 - lov
                m = plsc.bitcast(dl16, jnp.uint32) < rngv
                mi = m.astype(jnp.int32)
                pos = plsc.cumsum(mi) - mi + cntv
                pos = jnp.minimum(pos, CAP - 1)
                plsc.store_scatter(dl_list, [pos], dl16, mask=m)
                plsc.store_scatter(s_list, [pos], sscans[b][sl], mask=m)
                plsc.store_scatter(v_list, [pos], vscans[b][sl], mask=m)
                return cntv + plsc.all_reduce_population_count(m)
            return plsc.parallel_loop(0, SCH // 16, unroll=6,
                                      carry=cntv0)(scan_vec)

        scan_issue(0, 0)

        def scan_pair(ci2, cntv):
            ci = ci2 * 2

            @pl.when(ci + 1 < NSCAN)
            def _():
                scan_issue(ci + 1, 1)
            scan_drain(ci, 0)
            cntv = scan_compute(0, cntv)

            @pl.when(ci + 2 < NSCAN)
            def _():
                scan_issue(ci + 2, 0)
            scan_drain(ci + 1, 1)
            cntv = scan_compute(1, cntv)
            return cntv

        cntv = lax.fori_loop(0, NSCAN // 2, scan_pair,
                             jnp.zeros((16,), jnp.int32))

        for kk in range(2 * CH // 16):
            addr = jnp.minimum(cntv + iota + kk * 16, CAP - 1)
            zi = jnp.zeros((16,), jnp.int32)
            plsc.store_scatter(dl_list, [addr], zi, mask=None)
            plsc.store_scatter(s_list, [addr], zi, mask=None)
            plsc.store_scatter(v_list, [addr], zeros16, mask=None)

        cnt = jnp.max(cntv)
        nch = (cnt + CH - 1) // CH
        nch2 = 2 * ((nch + 1) // 2)
        lastb = jnp.maximum(nch2 - 1, 0) * CH

        def p2_issue(base, b):
            pltpu.async_copy(feat_hbm.at[s_list.at[pl.dslice(base, CH)]],
                             rowss[b], sems[b])

        def p2_drain(base, b):
            pltpu.make_async_copy(feat_hbm.at[s_list.at[pl.dslice(base, CH)]],
                                  rowss[b], sems[b]).wait()

        def p2_compute(base, b):
            cntb = jnp.full((16,), 0, jnp.int32) + (cnt - base)

            def grp_body(g):
                gsl = pl.ds(g * 16, 16)
                vv16 = v_list[pl.ds(base + g * 16, 16)]
                vv16 = jnp.where(g * 16 + iota < cntb, vv16, zeros16)
                dl16 = dl_list[pl.ds(base + g * 16, 16)]
                for e16 in range(16):
                    vv = jnp.full((16,), vv16[e16], jnp.float32)
                    dlv = jnp.full((16,), dl16[e16], jnp.int32)
                    e = g * 16 + e16
                    for d in range(D // 16):
                        csl = pl.ds(d * 16, 16)
                        plsc.addupdate_scatter(
                            acc, [dlv, iota + d * 16], rowss[b][e, csl] * vv)
            plsc.parallel_loop(0, CH // 16)(grp_body)

        p2_issue(0, 0)

        def p2_pair(i2, _):
            base = i2 * 2 * CH
            p2_issue(jnp.minimum(base + CH, lastb), 1)
            p2_drain(base, 0)
            p2_compute(base, 0)
            p2_issue(jnp.minimum(base + 2 * CH, lastb), 0)
            p2_drain(jnp.minimum(base + CH, lastb), 1)
            p2_compute(base + CH, 1)
            return _
        lax.fori_loop(0, nch2 // 2, p2_pair, None)
        p2_drain(lastb, 0)

        pltpu.sync_copy(acc, out_hbm.at[pl.ds(lo, RANGE)])

    return k(features, src, dst, vals)


def _tc_combine(features, lap, W1, b1, W2, b2):
    BN = 1000
    bias = (b1 + b2).reshape(1, D)

    def body(f_ref, l_ref, w1_ref, w2_ref, b_ref, o_ref):
        lap_b = l_ref[...]
        f = f_ref[...]
        m1 = lap_b + f
        m2 = lap_b * f
        dn = (((1,), (1,)), ((), ()))
        o_ref[...] = (
            lax.dot_general(m1, w1_ref[...], dn,
                            preferred_element_type=jnp.float32)
            + lax.dot_general(m2, w2_ref[...], dn,
                              preferred_element_type=jnp.float32)
            + b_ref[...]
        )

    row_spec = pl.BlockSpec((BN, D), lambda i: (i, 0))
    full_spec = pl.BlockSpec((D, D), lambda i: (0, 0))
    return pl.pallas_call(
        body,
        grid=(N // BN,),
        in_specs=[row_spec, row_spec, full_spec, full_spec,
                  pl.BlockSpec((1, D), lambda i: (0, 0))],
        out_specs=row_spec,
        out_shape=jax.ShapeDtypeStruct((N, D), jnp.float32),
    )(features, lap, W1, W2, bias)


@jax.jit
def kernel(features, edge_index, edge_vals, W1, b1, W2, b2):
    dst = edge_index[0]
    src = edge_index[1]
    lap_pad = _sc_spmm(features, src, dst, edge_vals)
    return _tc_combine(features, lap_pad[:N], W1, b1, W2, b2)


# speedup vs baseline: 1.8650x; 1.8650x over previous
import functools

import jax
import jax.numpy as jnp
from jax import lax
from jax.experimental import pallas as pl
from jax.experimental.pallas import tpu as pltpu
from jax.experimental.pallas import tpu_sc as plsc

N = 10000
E = 320000
D = 128
NC = 2
NS = 16
NW = NC * NS
CH = 128
NCHUNKS = E // CH
BASE = NCHUNKS // NW
EXTRA = NCHUNKS - BASE * NW
ZR = 48
RPS = 624
TAIL = N - NS * RPS


def _sc_spmm(features, src, dst, vals):
    mesh = plsc.VectorSubcoreMesh(core_axis_name="c", subcore_axis_name="s")

    @functools.partial(
        pl.kernel,
        out_type=jax.ShapeDtypeStruct((NC, N, D), jnp.float32),
        mesh=mesh,
        scratch_types=[
            pltpu.VMEM((CH,), jnp.int32),
            pltpu.VMEM((CH,), jnp.int32),
            pltpu.VMEM((CH,), jnp.int32),
            pltpu.VMEM((CH,), jnp.float32),
            pltpu.VMEM((CH, D), jnp.float32),
            pltpu.VMEM((CH, D), jnp.float32),
            pltpu.VMEM((ZR, D), jnp.float32),
            pltpu.VMEM_SHARED((N, D), jnp.float32),
            pltpu.SemaphoreType.DMA,
            pltpu.SemaphoreType.DMA,
            pltpu.SemaphoreType.DMA,
        ],
        compiler_params=pltpu.CompilerParams(needs_layout_passes=False),
    )
    def k(feat_hbm, src_hbm, dst_hbm, vals_hbm, out_hbm,
          src_v, dst0, dst1, vals_v, rows0, rows1, zbuf, acc,
          sem, ss0, ss1):
        dsts = (dst0, dst1)
        rows = (rows0, rows1)
        ssem = (ss0, ss1)
        c = lax.axis_index("c")
        s = lax.axis_index("s")
        wid = s * NC + c

        def zero_row(r, _):
            for d in range(D // 16):
                zbuf[r, pl.ds(d * 16, 16)] = jnp.zeros((16,), jnp.float32)
            return _
        lax.fori_loop(0, ZR, zero_row, None)
        for j in range(RPS // ZR):
            pltpu.sync_copy(zbuf, acc.at[pl.ds(s * RPS + j * ZR, ZR)])

        @pl.when(s == 0)
        def _():
            pltpu.sync_copy(zbuf.at[pl.ds(0, TAIL)],
                            acc.at[pl.ds(NS * RPS, TAIL)])
        plsc.subcore_barrier()

        def do_chunk(g, b, wait_prev):
            off = g * CH
            pltpu.sync_copy(src_hbm.at[pl.ds(off, CH)], src_v)
            pltpu.sync_copy(dst_hbm.at[pl.ds(off, CH)], dsts[b])
            pltpu.sync_copy(vals_hbm.at[pl.ds(off, CH)], vals_v)
            if wait_prev:
                pltpu.make_async_copy(rows[b], acc.at[dsts[b]],
                                      ssem[b]).wait()
            pltpu.async_copy(feat_hbm.at[src_v], rows[b], sem).wait()

            @plsc.parallel_loop(0, CH, unroll=4)
            def _(e):
                vv = plsc.load_gather(vals_v, [jnp.full((16,), e, jnp.int32)])
                for d in range(D // 16):
                    sl = pl.ds(d * 16, 16)
                    rows[b][e, sl] = rows[b][e, sl] * vv
            pltpu.async_copy(rows[b], acc.at[dsts[b]], ssem[b], add=True)

        do_chunk(wid, 0, False)
        do_chunk(NW + wid, 1, False)

        def chunk_pair(i2, _):
            i = 2 + 2 * i2
            do_chunk(i * NW + wid, 0, True)
            do_chunk((i + 1) * NW + wid, 1, True)
            return _
        lax.fori_loop(0, (BASE - 2) // 2, chunk_pair, None)

        @pl.when(wid < EXTRA)
        def _():
            do_chunk(BASE * NW + wid, 0, True)
            pltpu.make_async_copy(rows[0], acc.at[dsts[0]], ssem[0]).wait()
            pltpu.make_async_copy(rows[1], acc.at[dsts[1]], ssem[1]).wait()

        @pl.when(wid >= EXTRA)
        def _():
            pltpu.make_async_copy(rows[0], acc.at[dsts[0]], ssem[0]).wait()
            pltpu.make_async_copy(rows[1], acc.at[dsts[1]], ssem[1]).wait()

        plsc.subcore_barrier()
        for j in range(RPS // ZR):
            off = s * RPS + j * ZR
            pltpu.sync_copy(acc.at[pl.ds(off, ZR)],
                            out_hbm.at[c].at[pl.ds(off, ZR)])

        @pl.when(s == 0)
        def _():
            pltpu.sync_copy(acc.at[pl.ds(NS * RPS, TAIL)],
                            out_hbm.at[c].at[pl.ds(NS * RPS, TAIL)])

    return k(features, src, dst, vals)


def _tc_combine(features, partial, W1, b1, W2, b2):
    BN = 1000
    bias = (b1 + b2).reshape(1, D)
    p0 = partial[0]
    p1 = partial[1]

    def body(f_ref, p0_ref, p1_ref, w1_ref, w2_ref, b_ref, o_ref):
        lap = p0_ref[...] + p1_ref[...]
        f = f_ref[...]
        m1 = lap + f
        m2 = lap * f
        dn = (((1,), (1,)), ((), ()))
        o_ref[...] = (
            lax.dot_general(m1, w1_ref[...], dn,
                            preferred_element_type=jnp.float32)
            + lax.dot_general(m2, w2_ref[...], dn,
                              preferred_element_type=jnp.float32)
            + b_ref[...]
        )

    row_spec = pl.BlockSpec((BN, D), lambda i: (i, 0))
    full_spec = pl.BlockSpec((D, D), lambda i: (0, 0))
    return pl.pallas_call(
        body,
        grid=(N // BN,),
        in_specs=[row_spec, row_spec, row_spec, full_spec, full_spec,
                  pl.BlockSpec((1, D), lambda i: (0, 0))],
        out_specs=row_spec,
        out_shape=jax.ShapeDtypeStruct((N, D), jnp.float32),
    )(features, p0, p1, W1, W2, bias)


@jax.jit
def kernel(features, edge_index, edge_vals, W1, b1, W2, b2):
    dst = edge_index[0]
    src = edge_index[1]
    partial = _sc_spmm(features, src, dst, edge_vals)
    return _tc_combine(features, partial, W1, b1, W2, b2)


# speedup vs baseline: 2.3935x; 1.2834x over previous
import functools

import jax
import jax.numpy as jnp
from jax import lax
from jax.experimental import pallas as pl
from jax.experimental.pallas import tpu as pltpu
from jax.experimental.pallas import tpu_sc as plsc

N = 10000
E = 320000
D = 128
NC = 2
NS = 16
NW = NC * NS
CH = 128
NCHUNKS = E // CH
BASE = NCHUNKS // NW
EXTRA = NCHUNKS - BASE * NW
ZR = 48
RPS = 624
TAIL = N - NS * RPS


def _sc_spmm(features, edata):
    mesh = plsc.VectorSubcoreMesh(core_axis_name="c", subcore_axis_name="s")

    @functools.partial(
        pl.kernel,
        out_type=jax.ShapeDtypeStruct((NC, N, D), jnp.float32),
        mesh=mesh,
        scratch_types=[
            pltpu.VMEM((3, CH), jnp.int32),
            pltpu.VMEM((3, CH), jnp.int32),
            pltpu.VMEM((CH, D), jnp.float32),
            pltpu.VMEM((CH, D), jnp.float32),
            pltpu.VMEM((ZR, D), jnp.float32),
            pltpu.VMEM_SHARED((N, D), jnp.float32),
            pltpu.SemaphoreType.DMA,
            pltpu.SemaphoreType.DMA,
            pltpu.SemaphoreType.DMA,
        ],
        compiler_params=pltpu.CompilerParams(needs_layout_passes=False),
    )
    def k(feat_hbm, edata_hbm, out_hbm,
          ed0, ed1, rows0, rows1, zbuf, acc,
          sem, ss0, ss1):
        eds = (ed0, ed1)
        rows = (rows0, rows1)
        ssem = (ss0, ss1)
        c = lax.axis_index("c")
        s = lax.axis_index("s")
        wid = s * NC + c

        def zero_row(r, _):
            for d in range(D // 16):
                zbuf[r, pl.ds(d * 16, 16)] = jnp.zeros((16,), jnp.float32)
            return _
        lax.fori_loop(0, ZR, zero_row, None)
        for j in range(RPS // ZR):
            pltpu.sync_copy(zbuf, acc.at[pl.ds(s * RPS + j * ZR, ZR)])

        @pl.when(s == 0)
        def _():
            pltpu.sync_copy(zbuf.at[pl.ds(0, TAIL)],
                            acc.at[pl.ds(NS * RPS, TAIL)])
        plsc.subcore_barrier()

        def do_chunk(g, b, wait_prev):
            if wait_prev:
                pltpu.make_async_copy(rows[b], acc.at[eds[b].at[1]],
                                      ssem[b]).wait()
            pltpu.sync_copy(edata_hbm.at[g], eds[b])
            pltpu.async_copy(feat_hbm.at[eds[b].at[0]], rows[b], sem).wait()

            @plsc.parallel_loop(0, CH, unroll=8)
            def _(e):
                vi = plsc.load_gather(eds[b].at[2],
                                      [jnp.full((16,), e, jnp.int32)])
                vv = plsc.bitcast(vi, jnp.float32)
                for d in range(D // 16):
                    sl = pl.ds(d * 16, 16)
                    rows[b][e, sl] = rows[b][e, sl] * vv
            pltpu.async_copy(rows[b], acc.at[eds[b].at[1]], ssem[b],
                             add=True)

        do_chunk(wid, 0, False)
        do_chunk(NW + wid, 1, False)

        def chunk_pair(i2, _):
            i = 2 + 2 * i2
            do_chunk(i * NW + wid, 0, True)
            do_chunk((i + 1) * NW + wid, 1, True)
            return _
        lax.fori_loop(0, (BASE - 2) // 2, chunk_pair, None)

        @pl.when(wid < EXTRA)
        def _():
            do_chunk(BASE * NW + wid, 0, True)
            pltpu.make_async_copy(rows[0], acc.at[eds[0].at[1]],
                                  ssem[0]).wait()
            pltpu.make_async_copy(rows[1], acc.at[eds[1].at[1]],
                                  ssem[1]).wait()

        @pl.when(wid >= EXTRA)
        def _():
            pltpu.make_async_copy(rows[0], acc.at[eds[0].at[1]],
                                  ssem[0]).wait()
            pltpu.make_async_copy(rows[1], acc.at[eds[1].at[1]],
                                  ssem[1]).wait()

        plsc.subcore_barrier()
        for j in range(RPS // ZR):
            off = s * RPS + j * ZR
            pltpu.sync_copy(acc.at[pl.ds(off, ZR)],
                            out_hbm.at[c].at[pl.ds(off, ZR)])

        @pl.when(s == 0)
        def _():
            pltpu.sync_copy(acc.at[pl.ds(NS * RPS, TAIL)],
                            out_hbm.at[c].at[pl.ds(NS * RPS, TAIL)])

    return k(features, edata)


def _tc_combine(features, partial, W1, b1, W2, b2):
    BN = 1000
    bias = (b1 + b2).reshape(1, D)
    p0 = partial[0]
    p1 = partial[1]

    def body(f_ref, p0_ref, p1_ref, w1_ref, w2_ref, b_ref, o_ref):
        lap = p0_ref[...] + p1_ref[...]
        f = f_ref[...]
        m1 = lap + f
        m2 = lap * f
        dn = (((1,), (1,)), ((), ()))
        o_ref[...] = (
            lax.dot_general(m1, w1_ref[...], dn,
                            preferred_element_type=jnp.float32)
            + lax.dot_general(m2, w2_ref[...], dn,
                              preferred_element_type=jnp.float32)
            + b_ref[...]
        )

    row_spec = pl.BlockSpec((BN, D), lambda i: (i, 0))
    full_spec = pl.BlockSpec((D, D), lambda i: (0, 0))
    return pl.pallas_call(
        body,
        grid=(N // BN,),
        in_specs=[row_spec, row_spec, row_spec, full_spec, full_spec,
                  pl.BlockSpec((1, D), lambda i: (0, 0))],
        out_specs=row_spec,
        out_shape=jax.ShapeDtypeStruct((N, D), jnp.float32),
    )(features, p0, p1, W1, W2, bias)


@jax.jit
def kernel(features, edge_index, edge_vals, W1, b1, W2, b2):
    dst = edge_index[0]
    src = edge_index[1]
    vals_i = lax.bitcast_convert_type(edge_vals, jnp.int32)
    edata = jnp.stack([src.reshape(NCHUNKS, CH), dst.reshape(NCHUNKS, CH),
                       vals_i.reshape(NCHUNKS, CH)], axis=1)
    partial = _sc_spmm(features, edata)
    return _tc_combine(features, partial, W1, b1, W2, b2)
